# initial kernel scaffold (unmeasured)
import jax
import jax.numpy as jnp
from jax import lax
from jax.experimental import pallas as pl
from jax.experimental.pallas import tpu as pltpu


def kernel(
    x,
):
    def body(*refs):
        pass

    out_shape = jax.ShapeDtypeStruct(..., jnp.float32)
    return pl.pallas_call(body, out_shape=out_shape)(...)


# baseline (device time: 1622570 ns/iter reference)
import jax
import jax.numpy as jnp
from jax import lax
from jax.experimental import pallas as pl
from jax.experimental.pallas import tpu as pltpu

CHUNK = 1024


def kernel(x):
    m, n = x.shape
    steps = m // CHUNK

    def body(x_ref, out_ref, recv_buf, send_sem, recv_sem, credit_sem):
        step = pl.program_id(0)
        my_x = lax.axis_index("x")
        my_y = lax.axis_index("y")
        my_z = lax.axis_index("z")
        partner = (1 - my_x, my_y, my_z)

        @pl.when(step == 0)
        def _():
            barrier_sem = pltpu.get_barrier_semaphore()
            pl.semaphore_signal(
                barrier_sem, inc=1,
                device_id=partner, device_id_type=pl.DeviceIdType.MESH,
            )
            pl.semaphore_wait(barrier_sem, 1)

        @pl.when(step > 0)
        def _():
            pl.semaphore_wait(credit_sem, 1)

        rdma = pltpu.make_async_remote_copy(
            src_ref=x_ref,
            dst_ref=recv_buf,
            send_sem=send_sem,
            recv_sem=recv_sem,
            device_id=partner,
            device_id_type=pl.DeviceIdType.MESH,
        )
        rdma.start()
        rdma.wait()

        out_ref[...] = x_ref[...] + recv_buf[...]

        @pl.when(step < steps - 1)
        def _():
            pl.semaphore_signal(
                credit_sem, inc=1,
                device_id=partner, device_id_type=pl.DeviceIdType.MESH,
            )

    return pl.pallas_call(
        body,
        grid=(steps,),
        in_specs=[pl.BlockSpec((CHUNK, n), lambda i: (i, 0))],
        out_specs=pl.BlockSpec((CHUNK, n), lambda i: (i, 0)),
        out_shape=jax.ShapeDtypeStruct((m, n), x.dtype),
        scratch_shapes=[
            pltpu.VMEM((CHUNK, n), x.dtype),
            pltpu.SemaphoreType.DMA,
            pltpu.SemaphoreType.DMA,
            pltpu.SemaphoreType.REGULAR,
        ],
        compiler_params=pltpu.CompilerParams(collective_id=0),
    )(x)


# device time: 933986 ns/iter; 1.7373x vs baseline; 1.7373x over previous
import jax
import jax.numpy as jnp
from jax import lax
from jax.experimental import pallas as pl
from jax.experimental.pallas import tpu as pltpu

C = 512
B = 2 * C


def kernel(x):
    m, n = x.shape
    steps = m // B

    def body(x_hbm, x_prev, out_ref, recvx, recvy,
             x_send_sem, x_recv_sem, y_send_sem, y_recv_sem,
             x_credit, y_credit):
        i = pl.program_id(0)
        my_x = lax.axis_index("x")
        my_y = lax.axis_index("y")
        my_z = lax.axis_index("z")
        xp = (1 - my_x, my_y, my_z)
        yp = (my_x, 1 - my_y, my_z)

        def x_rdma(k):
            slot = k % 2
            return pltpu.make_async_remote_copy(
                src_ref=x_hbm.at[pl.ds(k * B + my_y * C, C), :],
                dst_ref=recvx.at[slot],
                send_sem=x_send_sem.at[slot],
                recv_sem=x_recv_sem.at[slot],
                device_id=xp,
                device_id_type=pl.DeviceIdType.MESH,
            )

        def y_rdma(k):
            slot = k % 2
            return pltpu.make_async_remote_copy(
                src_ref=recvx.at[slot],
                dst_ref=recvy.at[slot],
                send_sem=y_send_sem.at[slot],
                recv_sem=y_recv_sem.at[slot],
                device_id=yp,
                device_id_type=pl.DeviceIdType.MESH,
            )

        @pl.when(i == 0)
        def _():
            bar = pltpu.get_barrier_semaphore()
            for nbr in (xp, yp):
                pl.semaphore_signal(
                    bar, inc=1,
                    device_id=nbr, device_id_type=pl.DeviceIdType.MESH,
                )
            pl.semaphore_wait(bar, 2)

        @pl.when(i < steps)
        def _():
            @pl.when(i >= 2)
            def _():
                pl.semaphore_wait(x_credit, 1)
            x_rdma(i).start()

        @pl.when(i >= 1)
        def _():
            j = i - 1
            slot = j % 2
            xr = x_rdma(j)
            xr.wait_recv()
            xr.wait_send()

            @pl.when(j >= 2)
            def _():
                pl.semaphore_wait(y_credit, 1)
            yr = y_rdma(j)
            yr.start()
            yr.wait_recv()

            out_ref[pl.ds(my_y * C, C), :] = (
                x_prev[pl.ds(my_y * C, C), :] + recvx[slot])
            out_ref[pl.ds((1 - my_y) * C, C), :] = (
                x_prev[pl.ds((1 - my_y) * C, C), :] + recvy[slot])

            yr.wait_send()

            @pl.when(j <= steps - 3)
            def _():
                pl.semaphore_signal(
                    x_credit, inc=1,
                    device_id=xp, device_id_type=pl.DeviceIdType.MESH,
                )
                pl.semaphore_signal(
                    y_credit, inc=1,
                    device_id=yp, device_id_type=pl.DeviceIdType.MESH,
                )

    lagged = lambda i: (jnp.maximum(i - 1, 0), 0)
    return pl.pallas_call(
        body,
        grid=(steps + 1,),
        in_specs=[
            pl.BlockSpec(memory_space=pl.ANY),
            pl.BlockSpec((B, n), lagged),
        ],
        out_specs=pl.BlockSpec((B, n), lagged),
        out_shape=jax.ShapeDtypeStruct((m, n), x.dtype),
        scratch_shapes=[
            pltpu.VMEM((2, C, n), x.dtype),
            pltpu.VMEM((2, C, n), x.dtype),
            pltpu.SemaphoreType.DMA((2,)),
            pltpu.SemaphoreType.DMA((2,)),
            pltpu.SemaphoreType.DMA((2,)),
            pltpu.SemaphoreType.DMA((2,)),
            pltpu.SemaphoreType.REGULAR,
            pltpu.SemaphoreType.REGULAR,
        ],
        compiler_params=pltpu.CompilerParams(collective_id=0),
    )(x, x)


# device time: 918669 ns/iter; 1.7662x vs baseline; 1.0167x over previous
import jax
import jax.numpy as jnp
from jax import lax
from jax.experimental import pallas as pl
from jax.experimental.pallas import tpu as pltpu

C = 512
B = 2 * C
S = 3


def kernel(x):
    m, n = x.shape
    steps = m // B

    def body(x_hbm, x_prev, out_ref, recvx, recvy,
             x_send_sem, x_recv_sem, y_send_sem, y_recv_sem,
             x_credit, y_credit):
        i = pl.program_id(0)
        my_x = lax.axis_index("x")
        my_y = lax.axis_index("y")
        my_z = lax.axis_index("z")
        xp = (1 - my_x, my_y, my_z)
        yp = (my_x, 1 - my_y, my_z)

        def x_rdma(k):
            slot = k % S
            return pltpu.make_async_remote_copy(
                src_ref=x_hbm.at[pl.ds(k * B + my_y * C, C), :],
                dst_ref=recvx.at[slot],
                send_sem=x_send_sem.at[slot],
                recv_sem=x_recv_sem.at[slot],
                device_id=xp,
                device_id_type=pl.DeviceIdType.MESH,
            )

        def y_rdma(k):
            slot = k % S
            return pltpu.make_async_remote_copy(
                src_ref=recvx.at[slot],
                dst_ref=recvy.at[slot],
                send_sem=y_send_sem.at[slot],
                recv_sem=y_recv_sem.at[slot],
                device_id=yp,
                device_id_type=pl.DeviceIdType.MESH,
            )

        @pl.when(i == 0)
        def _():
            bar = pltpu.get_barrier_semaphore()
            for nbr in (xp, yp):
                pl.semaphore_signal(
                    bar, inc=1,
                    device_id=nbr, device_id_type=pl.DeviceIdType.MESH,
                )
            pl.semaphore_wait(bar, 2)

        @pl.when(i < steps)
        def _():
            @pl.when(i >= S)
            def _():
                pl.semaphore_wait(x_credit, 1)
            x_rdma(i).start()

        @pl.when(i >= 1)
        def _():
            j = i - 1
            slot = j % S
            xr = x_rdma(j)
            xr.wait_recv()

            @pl.when(j >= S)
            def _():
                pl.semaphore_wait(y_credit, 1)
            yr = y_rdma(j)
            yr.start()

            out_ref[pl.ds(my_y * C, C), :] = (
                x_prev[pl.ds(my_y * C, C), :] + recvx[slot])
            xr.wait_send()
            yr.wait_recv()
            out_ref[pl.ds((1 - my_y) * C, C), :] = (
                x_prev[pl.ds((1 - my_y) * C, C), :] + recvy[slot])

            yr.wait_send()

            @pl.when(j <= steps - 1 - S)
            def _():
                pl.semaphore_signal(
                    x_credit, inc=1,
                    device_id=xp, device_id_type=pl.DeviceIdType.MESH,
                )
                pl.semaphore_signal(
                    y_credit, inc=1,
                    device_id=yp, device_id_type=pl.DeviceIdType.MESH,
                )

    lagged = lambda i: (jnp.maximum(i - 1, 0), 0)
    return pl.pallas_call(
        body,
        grid=(steps + 1,),
        in_specs=[
            pl.BlockSpec(memory_space=pl.ANY),
            pl.BlockSpec((B, n), lagged),
        ],
        out_specs=pl.BlockSpec((B, n), lagged),
        out_shape=jax.ShapeDtypeStruct((m, n), x.dtype),
        scratch_shapes=[
            pltpu.VMEM((S, C, n), x.dtype),
            pltpu.VMEM((S, C, n), x.dtype),
            pltpu.SemaphoreType.DMA((S,)),
            pltpu.SemaphoreType.DMA((S,)),
            pltpu.SemaphoreType.DMA((S,)),
            pltpu.SemaphoreType.DMA((S,)),
            pltpu.SemaphoreType.REGULAR,
            pltpu.SemaphoreType.REGULAR,
        ],
        compiler_params=pltpu.CompilerParams(collective_id=0),
    )(x, x)


# device time: 903375 ns/iter; 1.7961x vs baseline; 1.0169x over previous
import jax
import jax.numpy as jnp
from jax import lax
from jax.experimental import pallas as pl
from jax.experimental.pallas import tpu as pltpu

C = 512
B = 2 * C
S = 4
L = 2


def kernel(x):
    m, n = x.shape
    steps = m // B

    def body(x_hbm, x_prev, out_ref, recvx, recvy,
             x_send_sem, x_recv_sem, y_send_sem, y_recv_sem,
             x_credit, y_credit):
        i = pl.program_id(0)
        my_x = lax.axis_index("x")
        my_y = lax.axis_index("y")
        my_z = lax.axis_index("z")
        xp = (1 - my_x, my_y, my_z)
        yp = (my_x, 1 - my_y, my_z)

        def x_rdma(k):
            slot = k % S
            return pltpu.make_async_remote_copy(
                src_ref=x_hbm.at[pl.ds(k * B + my_y * C, C), :],
                dst_ref=recvx.at[slot],
                send_sem=x_send_sem.at[slot],
                recv_sem=x_recv_sem.at[slot],
                device_id=xp,
                device_id_type=pl.DeviceIdType.MESH,
            )

        def y_rdma(k):
            slot = k % S
            return pltpu.make_async_remote_copy(
                src_ref=recvx.at[slot],
                dst_ref=recvy.at[slot],
                send_sem=y_send_sem.at[slot],
                recv_sem=y_recv_sem.at[slot],
                device_id=yp,
                device_id_type=pl.DeviceIdType.MESH,
            )

        @pl.when(i == 0)
        def _():
            bar = pltpu.get_barrier_semaphore()
            for nbr in (xp, yp):
                pl.semaphore_signal(
                    bar, inc=1,
                    device_id=nbr, device_id_type=pl.DeviceIdType.MESH,
                )
            pl.semaphore_wait(bar, 2)

        @pl.when(i < steps)
        def _():
            @pl.when(i >= S)
            def _():
                pl.semaphore_wait(x_credit, 1)
            x_rdma(i).start()

        @pl.when(i >= L)
        def _():
            j = i - L
            slot = j % S
            xr = x_rdma(j)
            xr.wait_recv()

            @pl.when(j >= S)
            def _():
                pl.semaphore_wait(y_credit, 1)
            yr = y_rdma(j)
            yr.start()

            out_ref[pl.ds(my_y * C, C), :] = (
                x_prev[pl.ds(my_y * C, C), :] + recvx[slot])
            xr.wait_send()
            yr.wait_recv()
            out_ref[pl.ds((1 - my_y) * C, C), :] = (
                x_prev[pl.ds((1 - my_y) * C, C), :] + recvy[slot])

            yr.wait_send()

            @pl.when(j <= steps - 1 - S)
            def _():
                pl.semaphore_signal(
                    x_credit, inc=1,
                    device_id=xp, device_id_type=pl.DeviceIdType.MESH,
                )
                pl.semaphore_signal(
                    y_credit, inc=1,
                    device_id=yp, device_id_type=pl.DeviceIdType.MESH,
                )

    lagged = lambda i: (jnp.maximum(i - L, 0), 0)
    return pl.pallas_call(
        body,
        grid=(steps + L,),
        in_specs=[
            pl.BlockSpec(memory_space=pl.ANY),
            pl.BlockSpec((B, n), lagged),
        ],
        out_specs=pl.BlockSpec((B, n), lagged),
        out_shape=jax.ShapeDtypeStruct((m, n), x.dtype),
        scratch_shapes=[
            pltpu.VMEM((S, C, n), x.dtype),
            pltpu.VMEM((S, C, n), x.dtype),
            pltpu.SemaphoreType.DMA((S,)),
            pltpu.SemaphoreType.DMA((S,)),
            pltpu.SemaphoreType.DMA((S,)),
            pltpu.SemaphoreType.DMA((S,)),
            pltpu.SemaphoreType.REGULAR,
            pltpu.SemaphoreType.REGULAR,
        ],
        compiler_params=pltpu.CompilerParams(collective_id=0),
    )(x, x)
